# baseline (speedup 1.0000x reference)
import functools

import jax
import jax.numpy as jnp
from jax import lax
from jax.experimental import pallas as pl
from jax.experimental.pallas import tpu as pltpu
from jax.experimental.pallas import tpu_sc as plsc

_NC = 2
_NS = 16
_NW = _NC * _NS
_L = 16

_B = 4
_S = 8192
_D = 8
_N = _B * _S
_PER_W = _N // _NW
_ROWS_PER_W = _S // _PER_W and _S // _PER_W
_W_PER_ROW = _S // _PER_W
_UNROLL = 4


def _body(ids_hbm, we_hbm, wp_hbm, bp_hbm, out_hbm,
          idx_v, out_v, we_v, wp_v, bp_v, tbl_v, sem1, sem2):
    wid = lax.axis_index("s") * _NC + lax.axis_index("c")
    base = wid * _PER_W

    b_row = wid // _W_PER_ROW
    s_off = (wid % _W_PER_ROW) * _PER_W
    cp_ids = pltpu.make_async_copy(
        ids_hbm.at[b_row, pl.ds(s_off, _PER_W)], idx_v, sem1)
    cp_we = pltpu.make_async_copy(we_hbm, we_v, sem2)
    cp_wp = pltpu.make_async_copy(wp_hbm, wp_v, sem2)
    cp_bp = pltpu.make_async_copy(bp_hbm, bp_v, sem2)
    cp_ids.start()
    cp_we.start()
    cp_wp.start()
    cp_bp.start()
    cp_we.wait()
    cp_wp.wait()
    cp_bp.wait()

    lane = lax.iota(jnp.int32, _L)
    d_vec = lane & 7
    lane8 = lane * _D

    bias = plsc.load_gather(bp_v, [d_vec])
    k_splats = [jnp.full((_L,), k, jnp.int32) for k in range(_D)]
    p_ks = [plsc.load_gather(wp_v, [d_vec, k_splats[k]]) for k in range(_D)]
    for t in range(4):
        v_vec = (lane >> 3) + 2 * t
        acc = bias
        for k in range(_D):
            e = plsc.load_gather(we_v, [v_vec, k_splats[k]])
            acc = acc + e * p_ks[k]
        tbl_v[pl.ds(_L * t, _L)] = acc

    cp_ids.wait()

    _HV = _PER_W // (2 * _L)
    _HF = _HV * _L * _D

    def _half(h):
        @plsc.parallel_loop(h * _HV, (h + 1) * _HV, step=1, unroll=_UNROLL)
        def _loop(jj):
            ids16 = idx_v[pl.ds(jj * _L, _L)]
            rowbase = ids16 * _D
            obase = jj * (_L * _D) + lane8
            for dd in range(_D):
                vals = plsc.load_gather(tbl_v, [rowbase + dd])
                plsc.store_scatter(out_v, [obase + dd], vals)
        return pltpu.make_async_copy(
            out_v.at[pl.ds(h * _HF, _HF)],
            out_hbm.at[pl.ds(base * _D + h * _HF, _HF)],
            sem1,
        )

    cp0 = _half(0)
    cp0.start()
    cp1 = _half(1)
    cp1.start()
    cp0.wait()
    cp1.wait()


_sc_call = functools.partial(
    pl.kernel,
    mesh=plsc.VectorSubcoreMesh(core_axis_name="c", subcore_axis_name="s"),
    out_type=jax.ShapeDtypeStruct((_N * _D,), jnp.float32),
    scratch_types=[
        pltpu.VMEM((_PER_W,), jnp.int32),
        pltpu.VMEM((_PER_W * _D,), jnp.float32),
        pltpu.VMEM((_D, _D), jnp.float32),
        pltpu.VMEM((_D, _D), jnp.float32),
        pltpu.VMEM((_D,), jnp.float32),
        pltpu.VMEM((64,), jnp.float32),
        pltpu.SemaphoreType.DMA,
        pltpu.SemaphoreType.DMA,
    ],
    compiler_params=pltpu.CompilerParams(
        needs_layout_passes=False,
        disable_bounds_checks=True,
        disable_semaphore_checks=True,
        skip_device_barrier=True,
    ),
)(_body)


def kernel(input_ids, W_emb, W_proj, b_proj):
    out = _sc_call(
        input_ids.astype(jnp.int32),
        W_emb.astype(jnp.float32),
        W_proj.astype(jnp.float32),
        b_proj.astype(jnp.float32),
    )
    return out.reshape(_B, _S, _D)


# speedup vs baseline: 1.0007x; 1.0007x over previous
import functools

import jax
import jax.numpy as jnp
from jax import lax
from jax.experimental import pallas as pl
from jax.experimental.pallas import tpu as pltpu
from jax.experimental.pallas import tpu_sc as plsc

_NC = 2
_NS = 16
_NW = _NC * _NS
_L = 16

_B = 4
_S = 8192
_D = 8
_N = _B * _S
_PER_W = _N // _NW
_ROWS_PER_W = _S // _PER_W and _S // _PER_W
_W_PER_ROW = _S // _PER_W
_UNROLL = 2


def _body(ids_hbm, we_hbm, wp_hbm, bp_hbm, out_hbm,
          idx_v, out_v, we_v, wp_v, bp_v, tbl_v, sem1, sem2):
    wid = lax.axis_index("s") * _NC + lax.axis_index("c")
    base = wid * _PER_W

    b_row = wid // _W_PER_ROW
    s_off = (wid % _W_PER_ROW) * _PER_W
    cp_ids = pltpu.make_async_copy(
        ids_hbm.at[b_row, pl.ds(s_off, _PER_W)], idx_v, sem1)
    cp_we = pltpu.make_async_copy(we_hbm, we_v, sem2)
    cp_wp = pltpu.make_async_copy(wp_hbm, wp_v, sem2)
    cp_bp = pltpu.make_async_copy(bp_hbm, bp_v, sem2)
    cp_we.start()
    cp_wp.start()
    cp_bp.start()
    cp_ids.start()
    cp_we.wait()
    cp_wp.wait()
    cp_bp.wait()

    lane = lax.iota(jnp.int32, _L)
    d_vec = lane & 7
    lane8 = lane * _D

    bias = plsc.load_gather(bp_v, [d_vec])
    k_splats = [jnp.full((_L,), k, jnp.int32) for k in range(_D)]
    p_ks = [plsc.load_gather(wp_v, [d_vec, k_splats[k]]) for k in range(_D)]
    for t in range(4):
        v_vec = (lane >> 3) + 2 * t
        acc = bias
        for k in range(_D):
            e = plsc.load_gather(we_v, [v_vec, k_splats[k]])
            acc = acc + e * p_ks[k]
        tbl_v[pl.ds(_L * t, _L)] = acc

    cp_ids.wait()

    _NQ = 4
    _HV = _PER_W // (_NQ * _L)
    _HF = _HV * _L * _D

    def _chunk(h):
        @plsc.parallel_loop(h * _HV, (h + 1) * _HV, step=1, unroll=_UNROLL)
        def _loop(jj):
            ids16 = idx_v[pl.ds(jj * _L, _L)]
            rowbase = ids16 * _D
            obase = jj * (_L * _D) + lane8
            for dd in range(_D):
                vals = plsc.load_gather(tbl_v, [rowbase + dd])
                plsc.store_scatter(out_v, [obase + dd], vals)
        return pltpu.make_async_copy(
            out_v.at[pl.ds(h * _HF, _HF)],
            out_hbm.at[pl.ds(base * _D + h * _HF, _HF)],
            sem1,
        )

    cps = []
    for h in range(_NQ):
        cp = _chunk(h)
        cp.start()
        cps.append(cp)
    for cp in cps:
        cp.wait()


_sc_call = functools.partial(
    pl.kernel,
    mesh=plsc.VectorSubcoreMesh(core_axis_name="c", subcore_axis_name="s"),
    out_type=jax.ShapeDtypeStruct((_N * _D,), jnp.float32),
    scratch_types=[
        pltpu.VMEM((_PER_W,), jnp.int32),
        pltpu.VMEM((_PER_W * _D,), jnp.float32),
        pltpu.VMEM((_D, _D), jnp.float32),
        pltpu.VMEM((_D, _D), jnp.float32),
        pltpu.VMEM((_D,), jnp.float32),
        pltpu.VMEM((64,), jnp.float32),
        pltpu.SemaphoreType.DMA,
        pltpu.SemaphoreType.DMA,
    ],
    compiler_params=pltpu.CompilerParams(
        needs_layout_passes=False,
        disable_bounds_checks=True,
        disable_semaphore_checks=True,
        skip_device_barrier=True,
    ),
)(_body)


def kernel(input_ids, W_emb, W_proj, b_proj):
    out = _sc_call(
        input_ids.astype(jnp.int32),
        W_emb.astype(jnp.float32),
        W_proj.astype(jnp.float32),
        b_proj.astype(jnp.float32),
    )
    return out.reshape(_B, _S, _D)


# speedup vs baseline: 1.0178x; 1.0170x over previous
import functools

import jax
import jax.numpy as jnp
from jax import lax
from jax.experimental import pallas as pl
from jax.experimental.pallas import tpu as pltpu
from jax.experimental.pallas import tpu_sc as plsc

_NC = 2
_NS = 16
_NW = _NC * _NS
_L = 16

_B = 4
_S = 8192
_D = 8
_N = _B * _S
_PER_W = _N // _NW
_ROWS_PER_W = _S // _PER_W and _S // _PER_W
_W_PER_ROW = _S // _PER_W
_UNROLL = 2


def _body(ids_hbm, we_hbm, wp_hbm, bp_hbm, out_hbm,
          idx_v, out_v, we_v, wp_v, bp_v, tbl_v, sem1, sem2):
    wid = lax.axis_index("s") * _NC + lax.axis_index("c")
    base = wid * _PER_W

    b_row = wid // _W_PER_ROW
    s_off = (wid % _W_PER_ROW) * _PER_W
    cp_ids = pltpu.make_async_copy(
        ids_hbm.at[b_row, pl.ds(s_off, _PER_W)], idx_v, sem1)
    cp_we = pltpu.make_async_copy(we_hbm, we_v, sem2)
    cp_wp = pltpu.make_async_copy(wp_hbm, wp_v, sem2)
    cp_bp = pltpu.make_async_copy(bp_hbm, bp_v, sem2)
    cp_we.start()
    cp_wp.start()
    cp_bp.start()
    cp_ids.start()
    cp_we.wait()
    cp_wp.wait()
    cp_bp.wait()

    lane = lax.iota(jnp.int32, _L)
    d_vec = lane & 7
    lane8 = lane * _D

    bias = plsc.load_gather(bp_v, [d_vec])
    k_splats = [jnp.full((_L,), k, jnp.int32) for k in range(_D)]
    p_ks = [plsc.load_gather(wp_v, [d_vec, k_splats[k]]) for k in range(_D)]
    for t in range(4):
        v_vec = (lane >> 3) + 2 * t
        acc = bias
        for k in range(_D):
            e = plsc.load_gather(we_v, [v_vec, k_splats[k]])
            acc = acc + e * p_ks[k]
        tbl_v[pl.ds(_L * t, _L)] = acc

    cp_ids.wait()

    _NQ = 2
    _HV = _PER_W // (_NQ * _L)
    _HF = _HV * _L * _D

    def _chunk(h):
        @plsc.parallel_loop(h * _HV, (h + 1) * _HV, step=1, unroll=_UNROLL)
        def _loop(jj):
            ids16 = idx_v[pl.ds(jj * _L, _L)]
            rowbase = ids16 * _D
            obase = jj * (_L * _D) + lane8
            for dd in range(_D):
                vals = plsc.load_gather(tbl_v, [rowbase + dd])
                plsc.store_scatter(out_v, [obase + dd], vals)
        return pltpu.make_async_copy(
            out_v.at[pl.ds(h * _HF, _HF)],
            out_hbm.at[pl.ds(base * _D + h * _HF, _HF)],
            sem1,
        )

    cps = []
    for h in range(_NQ):
        cp = _chunk(h)
        cp.start()
        cps.append(cp)
    for cp in cps:
        cp.wait()


_sc_call = functools.partial(
    pl.kernel,
    mesh=plsc.VectorSubcoreMesh(core_axis_name="c", subcore_axis_name="s"),
    out_type=jax.ShapeDtypeStruct((_N * _D,), jnp.float32),
    scratch_types=[
        pltpu.VMEM((_PER_W,), jnp.int32),
        pltpu.VMEM((_PER_W * _D,), jnp.float32),
        pltpu.VMEM((_D, _D), jnp.float32),
        pltpu.VMEM((_D, _D), jnp.float32),
        pltpu.VMEM((_D,), jnp.float32),
        pltpu.VMEM((64,), jnp.float32),
        pltpu.SemaphoreType.DMA,
        pltpu.SemaphoreType.DMA,
    ],
    compiler_params=pltpu.CompilerParams(
        needs_layout_passes=False,
        disable_bounds_checks=True,
        disable_semaphore_checks=True,
        skip_device_barrier=True,
    ),
)(_body)


def kernel(input_ids, W_emb, W_proj, b_proj):
    out = _sc_call(
        input_ids.astype(jnp.int32),
        W_emb.astype(jnp.float32),
        W_proj.astype(jnp.float32),
        b_proj.astype(jnp.float32),
    )
    return out.reshape(_B, _S, _D)
